# baseline (speedup 1.0000x reference)
import functools

import jax
import jax.numpy as jnp
from jax import lax
from jax.experimental import pallas as pl
from jax.experimental.pallas import tpu as pltpu
from jax.experimental.pallas import tpu_sc as plsc

B, L = 4096, 200
D_A, D_C = 64, 32
D_OUT = D_A + D_C
BL = B * L

K = 4
BK = B // K
BLC = BL // K

NC, NS = 2, 16
NW = NC * NS
CH = 128
PER_W = BLC // NW
NITER = PER_W // CH
IDXROWS = BL // CH
NBUF = 6
PF = 4
HEAD = 6
TAIL = 8
assert (NITER - HEAD - TAIL) % NBUF == 0 and PF < NBUF <= HEAD + (NBUF - PF)


def _make_sc_body(k):
    def _emb_body(aidx_hbm, cidx_hbm, emb_hbm, chem_hbm, out_hbm,
                  aidx_v, cidx_v, abuf, cbuf, gsems, wsems):
        wid = lax.axis_index("s") * NC + lax.axis_index("c")
        row0 = wid * PER_W
        it0 = k * (IDXROWS // K) + wid * NITER

        pltpu.sync_copy(aidx_hbm.at[pl.ds(it0, NITER)], aidx_v)
        pltpu.sync_copy(cidx_hbm.at[pl.ds(it0, NITER)], cidx_v)

        def gather_start(j, b):
            pltpu.async_copy(emb_hbm.at[aidx_v.at[j]], abuf.at[b], gsems.at[b])
            pltpu.async_copy(chem_hbm.at[cidx_v.at[j]], cbuf.at[b],
                             gsems.at[b])

        def gather_wait(b):
            pltpu.make_async_copy(emb_hbm.at[aidx_v.at[0]], abuf.at[b],
                                  gsems.at[b]).wait()
            pltpu.make_async_copy(chem_hbm.at[cidx_v.at[0]], cbuf.at[b],
                                  gsems.at[b]).wait()

        def write_start(j, b):
            r = row0 + j * CH
            pltpu.async_copy(abuf.at[b],
                             out_hbm.at[pl.ds(r, CH), pl.ds(0, D_A)],
                             wsems.at[b])
            pltpu.async_copy(cbuf.at[b],
                             out_hbm.at[pl.ds(r, CH), pl.ds(D_A, D_C)],
                             wsems.at[b])

        def write_wait(b):
            pltpu.make_async_copy(abuf.at[b],
                                  out_hbm.at[pl.ds(row0, CH), pl.ds(0, D_A)],
                                  wsems.at[b]).wait()
            pltpu.make_async_copy(cbuf.at[b],
                                  out_hbm.at[pl.ds(row0, CH),
                                             pl.ds(D_A, D_C)],
                                  wsems.at[b]).wait()

        def step(j, b, bn, wait_w, prefetch):
            gather_wait(b)
            write_start(j, b)
            if prefetch:
                if wait_w:
                    write_wait(bn)
                gather_start(j + PF, bn)

        for p in range(PF):
            gather_start(p, p % NBUF)

        for j in range(HEAD):
            step(j, j % NBUF, (j + PF) % NBUF, wait_w=(j >= NBUF - PF),
                 prefetch=True)

        @pl.loop(HEAD, NITER - TAIL, step=NBUF)
        def _main(g):
            for b in range(NBUF):
                step(g + b, b, (b + PF) % NBUF, wait_w=True, prefetch=True)

        for j in range(NITER - TAIL, NITER):
            step(j, j % NBUF, (j + PF) % NBUF, wait_w=True,
                 prefetch=(j + PF < NITER))

        for w in range(NITER - NBUF, NITER):
            write_wait(w % NBUF)

    return _emb_body


_sc_chunk = [
    functools.partial(
        pl.kernel,
        out_type=jax.ShapeDtypeStruct((BLC, 128), jnp.float32),
        mesh=plsc.VectorSubcoreMesh(core_axis_name="c", subcore_axis_name="s",
                                    num_cores=NC, num_subcores=NS),
        scratch_types=[
            pltpu.VMEM((NITER, CH), jnp.int32),
            pltpu.VMEM((NITER, CH), jnp.int32),
            pltpu.VMEM((NBUF, CH, D_A), jnp.float32),
            pltpu.VMEM((NBUF, CH, D_C), jnp.float32),
            pltpu.SemaphoreType.DMA((NBUF,)),
            pltpu.SemaphoreType.DMA((NBUF,)),
        ],
        compiler_params=pltpu.CompilerParams(use_tc_tiling_on_sc=False),
    )(_make_sc_body(k))
    for k in range(K)
]

BB = 64
GRID = BK // BB


def _fix_body(src_ref, dst_ref):
    dst_ref[...] = src_ref[:, :, :D_OUT]


def _fix_body_aliased(_, src_ref, dst_ref):
    dst_ref[...] = src_ref[:, :, :D_OUT]


def _tc_fix(k, chunk, partial=None):
    src_spec = pl.BlockSpec((BB, L, 128), lambda i: (i, 0, 0))
    dst_spec = pl.BlockSpec((BB, L, D_OUT), lambda i, _k=k: (_k * GRID + i, 0, 0))
    out_type = jax.ShapeDtypeStruct((B, L, D_OUT), jnp.float32)
    if partial is None:
        return pl.pallas_call(
            _fix_body, grid=(GRID,), in_specs=[src_spec],
            out_specs=dst_spec, out_shape=out_type)(chunk)
    return pl.pallas_call(
        _fix_body_aliased, grid=(GRID,),
        in_specs=[pl.BlockSpec(memory_space=pl.ANY), src_spec],
        out_specs=dst_spec, out_shape=out_type,
        input_output_aliases={0: 0})(partial, chunk)


def kernel(atom_types, chemistry_types, emb_table, chem_table):
    a = atom_types.reshape(IDXROWS, CH).astype(jnp.int32)
    c = chemistry_types.reshape(IDXROWS, CH).astype(jnp.int32)
    chunks = [_sc_chunk[k](a, c, emb_table, chem_table).reshape(BK, L, 128)
              for k in range(K)]
    out = _tc_fix(0, chunks[0])
    for k in range(1, K):
        out = _tc_fix(k, chunks[k], out)
    return out
